# Initial kernel scaffold; baseline (speedup 1.0000x reference)
import jax
import jax.numpy as jnp
from jax.experimental import pallas as pl


def kernel(x, edge_index, W1, b1, W2, b2):
    raise NotImplementedError("write your pallas kernel here")


# speedup vs baseline: 3.4979x; 3.4979x over previous
import functools

import jax
import jax.numpy as jnp
from jax import lax
from jax.experimental import pallas as pl
from jax.experimental.pallas import tpu as pltpu
import jax.experimental.pallas.tpu_sc as plsc

N = 10000
D = 128
HD = 64
E = 320000
NC = 2
NS = 16
EPT = E // NS
CH = 80
NCHUNK = EPT // CH
NP = 10112
RPT = NP // NS
BN = 1000


def _fill_f32(ref, rows, cols, value):
    v = jnp.full((16,), value, jnp.float32)

    @pl.loop(0, rows)
    def _(r):
        for c0 in range(cols // 16):
            ref[r, pl.ds(c0 * 16, 16)] = v


def _sc_body(with_deg, x_hbm, src_hbm, dst_hbm, *refs):
    if with_deg:
        (out_agg, out_deg, zagg, idx_src, idx_dst, rows, ones16, agg_sh,
         deg_sh, sem) = refs
    else:
        out_agg, zagg, idx_src, idx_dst, rows, agg_sh, sem = refs

    cid = lax.axis_index("c")
    sid = lax.axis_index("s")

    _fill_f32(zagg, RPT, HD, 0.0)
    pltpu.sync_copy(zagg, agg_sh.at[pl.ds(sid * RPT, RPT)])
    if with_deg:
        _fill_f32(ones16, RPT, 16, 0.0)

        @pl.when(cid == 0)
        def _():
            pltpu.sync_copy(ones16, deg_sh.at[pl.ds(sid * RPT, RPT)])

        _fill_f32(ones16, CH, 16, 1.0)
    plsc.subcore_barrier()

    @pl.loop(0, NCHUNK)
    def _(k):
        base = sid * EPT + k * CH
        pltpu.sync_copy(src_hbm.at[pl.ds(base, CH)], idx_src)
        pltpu.sync_copy(dst_hbm.at[pl.ds(base, CH)], idx_dst)

        @pl.loop(0, CH // 16)
        def _(j):
            v = idx_src[pl.ds(j * 16, 16)]
            idx_src[pl.ds(j * 16, 16)] = v * 2 + cid

        pltpu.async_copy(x_hbm.at[idx_src], rows, sem).wait()
        pltpu.sync_copy(rows, agg_sh.at[idx_dst], add=True)
        if with_deg:
            @pl.when(cid == 0)
            def _():
                pltpu.sync_copy(ones16.at[pl.ds(0, CH)], deg_sh.at[idx_dst],
                                add=True)

    plsc.subcore_barrier()
    pltpu.sync_copy(agg_sh.at[pl.ds(sid * RPT, RPT)],
                    out_agg.at[cid, pl.ds(sid * RPT, RPT)])
    if with_deg:
        @pl.when(cid == 0)
        def _():
            pltpu.sync_copy(deg_sh.at[pl.ds(sid * RPT, RPT)],
                            out_deg.at[pl.ds(sid * RPT, RPT)])


def _sc_segment_sum(x2, src, dst, with_deg):
    mesh = plsc.VectorSubcoreMesh(core_axis_name="c", subcore_axis_name="s")
    out_type = [jax.ShapeDtypeStruct((NC, NP, HD), jnp.float32)]
    scratch = [
        pltpu.VMEM((RPT, HD), jnp.float32),
        pltpu.VMEM((CH,), jnp.int32),
        pltpu.VMEM((CH,), jnp.int32),
        pltpu.VMEM((CH, HD), jnp.float32),
    ]
    if with_deg:
        out_type.append(jax.ShapeDtypeStruct((NP, 16), jnp.float32))
        scratch.append(pltpu.VMEM((max(RPT, CH), 16), jnp.float32))
        scratch.append(pltpu.VMEM_SHARED((NP, HD), jnp.float32))
        scratch.append(pltpu.VMEM_SHARED((NP, 16), jnp.float32))
    else:
        scratch.append(pltpu.VMEM_SHARED((NP, HD), jnp.float32))
    scratch.append(pltpu.SemaphoreType.DMA)
    fn = pl.kernel(
        functools.partial(_sc_body, with_deg),
        out_type=out_type,
        mesh=mesh,
        scratch_types=scratch,
        compiler_params=pltpu.CompilerParams(use_tc_tiling_on_sc=False),
    )
    return fn(x2, src, dst)


def _mlp_body(relu, p_ref, dg_ref, w_ref, b_ref, o_ref):
    s = jnp.concatenate([p_ref[0], p_ref[1]], axis=1)
    deg = jnp.maximum(jnp.max(dg_ref[...], axis=1, keepdims=True), 1.0)
    agg = s / deg
    y = jnp.dot(agg, w_ref[...], preferred_element_type=jnp.float32) + b_ref[...]
    if relu:
        y = jnp.maximum(y, 0.0)
    o_ref[...] = y


def _mlp(partials, degp, w, b, relu):
    return pl.pallas_call(
        functools.partial(_mlp_body, relu),
        grid=(N // BN,),
        in_specs=[
            pl.BlockSpec((NC, BN, HD), lambda i: (0, i, 0)),
            pl.BlockSpec((BN, 16), lambda i: (i, 0)),
            pl.BlockSpec((128, 128), lambda i: (0, 0)),
            pl.BlockSpec((1, 128), lambda i: (0, 0)),
        ],
        out_specs=pl.BlockSpec((BN, 128), lambda i: (i, 0)),
        out_shape=jax.ShapeDtypeStruct((N, 128), jnp.float32),
    )(partials, degp, w, b)


def kernel(x, edge_index, W1, b1, W2, b2):
    src = edge_index[0]
    dst = edge_index[1]
    p1, degp = _sc_segment_sum(x.reshape(2 * N, HD), src, dst, with_deg=True)
    h = _mlp(p1, degp, W1, b1.reshape(1, 128), relu=True)
    (p2,) = _sc_segment_sum(h.reshape(2 * N, HD), src, dst, with_deg=False)
    out = _mlp(p2, degp, W2, b2.reshape(1, 128), relu=False)
    return out


# speedup vs baseline: 9.9940x; 2.8572x over previous
import functools

import jax
import jax.numpy as jnp
from jax import lax
from jax.experimental import pallas as pl
from jax.experimental.pallas import tpu as pltpu
import jax.experimental.pallas.tpu_sc as plsc

N = 10000
D = 128
HD = 64
E = 320000
NC = 2
NS = 16
EPT = E // NS
CH = 80
NCHUNK = EPT // CH
RPT = N // NS
ZR = 125
BN = 1000


def _sc_body(with_deg, x_hbm, src_hbm, dst_hbm, *refs):
    if with_deg:
        (out_agg, out_deg, zagg, s1d, d1d, dstage, rows0, rows1, degloc,
         agg_sh, sg0, sg1, ss0, ss1) = refs
    else:
        (out_agg, zagg, s1d, d1d, dstage, rows0, rows1,
         agg_sh, sg0, sg1, ss0, ss1) = refs
        degloc = None

    cid = lax.axis_index("c")
    sid = lax.axis_index("s")

    pltpu.sync_copy(src_hbm.at[pl.ds(sid * EPT, EPT)], s1d)
    pltpu.sync_copy(dst_hbm.at[pl.ds(sid * EPT, EPT)], d1d)

    @pl.loop(0, EPT // 16, unroll=8)
    def _(i):
        v = s1d[pl.ds(i * 16, 16)]
        s1d[pl.ds(i * 16, 16)] = v * 2 + cid

    @pl.loop(0, NCHUNK, unroll=2)
    def _(r):
        for j in range(CH // 16):
            dstage[r, pl.ds(j * 16, 16)] = d1d[pl.ds(r * CH + j * 16, 16)]

    z = jnp.zeros((16,), jnp.float32)

    @pl.loop(0, ZR)
    def _(r):
        for c0 in range(HD // 16):
            zagg[r, pl.ds(c0 * 16, 16)] = z

    for q in range(RPT // ZR):
        pltpu.sync_copy(zagg, agg_sh.at[pl.ds(sid * RPT + q * ZR, ZR)])
    if with_deg:
        @pl.when(cid == 0)
        def _():
            @pl.loop(0, N // 16, unroll=4)
            def _(r):
                degloc[pl.ds(r * 16, 16)] = jnp.zeros((16,), jnp.float32)

    plsc.subcore_barrier()

    ones_v = jnp.ones((16,), jnp.float32)

    def fire_gather(k, buf, sem):
        pltpu.async_copy(x_hbm.at[s1d.at[pl.ds(k * CH, CH)]], buf, sem)

    def wait_gather(buf, sem):
        pltpu.make_async_copy(x_hbm.at[s1d.at[pl.ds(0, CH)]], buf, sem).wait()

    def halfstep(k, buf, sg, ss, fire_next):
        wait_gather(buf, sg)
        d = pltpu.async_copy(buf, agg_sh.at[dstage.at[k]], ss, add=True)
        if with_deg:
            @pl.when(cid == 0)
            def _():
                for j in range(CH // 16):
                    idx = dstage[k, pl.ds(j * 16, 16)]
                    plsc.addupdate_scatter(degloc, [idx], ones_v)
        d.wait()
        if fire_next:
            fire_gather(k + 2, buf, sg)

    fire_gather(0, rows0, sg0)
    fire_gather(1, rows1, sg1)

    @pl.loop(0, NCHUNK // 2 - 1)
    def _(g):
        k0 = g * 2
        halfstep(k0, rows0, sg0, ss0, True)
        halfstep(k0 + 1, rows1, sg1, ss1, True)

    halfstep(NCHUNK - 2, rows0, sg0, ss0, False)
    halfstep(NCHUNK - 1, rows1, sg1, ss1, False)

    plsc.subcore_barrier()
    pltpu.sync_copy(agg_sh.at[pl.ds(sid * RPT, RPT)],
                    out_agg.at[cid, pl.ds(sid * RPT, RPT)])
    if with_deg:
        @pl.when(cid == 0)
        def _():
            pltpu.sync_copy(degloc, out_deg.at[sid])


def _sc_segment_sum(x2, src, dst, with_deg):
    mesh = plsc.VectorSubcoreMesh(core_axis_name="c", subcore_axis_name="s")
    out_type = [jax.ShapeDtypeStruct((NC, N, HD), jnp.float32)]
    scratch = [
        pltpu.VMEM((ZR, HD), jnp.float32),
        pltpu.VMEM((EPT,), jnp.int32),
        pltpu.VMEM((EPT,), jnp.int32),
        pltpu.VMEM((NCHUNK, CH), jnp.int32),
        pltpu.VMEM((CH, HD), jnp.float32),
        pltpu.VMEM((CH, HD), jnp.float32),
    ]
    if with_deg:
        out_type.append(jax.ShapeDtypeStruct((NS, N), jnp.float32))
        scratch.append(pltpu.VMEM((N,), jnp.float32))
    scratch.append(pltpu.VMEM_SHARED((N, HD), jnp.float32))
    scratch.extend([pltpu.SemaphoreType.DMA] * 4)
    fn = pl.kernel(
        functools.partial(_sc_body, with_deg),
        out_type=out_type,
        mesh=mesh,
        scratch_types=scratch,
        compiler_params=pltpu.CompilerParams(use_tc_tiling_on_sc=False,
                                             needs_layout_passes=False),
    )
    return fn(x2, src, dst)


def _deg_body(dg_ref, o_ref):
    o_ref[...] = 1.0 / jnp.maximum(jnp.sum(dg_ref[...], axis=0), 1.0)[:, None]


def _deg_recip(degp):
    return pl.pallas_call(
        _deg_body,
        out_shape=jax.ShapeDtypeStruct((N, 1), jnp.float32),
    )(degp)


def _mlp_body(relu, p_ref, di_ref, w_ref, b_ref, o_ref):
    s = jnp.concatenate([p_ref[0], p_ref[1]], axis=1)
    agg = s * di_ref[...]
    y = jnp.dot(agg, w_ref[...], preferred_element_type=jnp.float32) + b_ref[...]
    if relu:
        y = jnp.maximum(y, 0.0)
    o_ref[...] = y


def _mlp(partials, dinv, w, b, relu):
    return pl.pallas_call(
        functools.partial(_mlp_body, relu),
        grid=(N // BN,),
        in_specs=[
            pl.BlockSpec((NC, BN, HD), lambda i: (0, i, 0)),
            pl.BlockSpec((BN, 1), lambda i: (i, 0)),
            pl.BlockSpec((128, 128), lambda i: (0, 0)),
            pl.BlockSpec((1, 128), lambda i: (0, 0)),
        ],
        out_specs=pl.BlockSpec((BN, 128), lambda i: (i, 0)),
        out_shape=jax.ShapeDtypeStruct((N, 128), jnp.float32),
    )(partials, dinv, w, b)


def kernel(x, edge_index, W1, b1, W2, b2):
    src = edge_index[0]
    dst = edge_index[1]
    p1, degp = _sc_segment_sum(x.reshape(2 * N, HD), src, dst, with_deg=True)
    dinv = _deg_recip(degp)
    h = _mlp(p1, dinv, W1, b1.reshape(1, 128), relu=True)
    (p2,) = _sc_segment_sum(h.reshape(2 * N, HD), src, dst, with_deg=False)
    out = _mlp(p2, dinv, W2, b2.reshape(1, 128), relu=False)
    return out
